# baseline (speedup 1.0000x reference)
import functools

import jax
import jax.numpy as jnp
from jax import lax
from jax.experimental import pallas as pl
from jax.experimental.pallas import tpu as pltpu
from jax.experimental.pallas import tpu_sc as plsc

MAXLEN = 200
EMBED_DIM = 64
BATCH = 4096

NUM_WORKERS = 32
IDX_MINOR = 100
ROWS_PER_CHUNK = 800
IDX_ROWS = ROWS_PER_CHUNK // IDX_MINOR
N_FLAT = BATCH * MAXLEN
CHUNKS_TOTAL = N_FLAT // ROWS_PER_CHUNK
CHUNKS_PER_WORKER = CHUNKS_TOTAL // NUM_WORKERS


@functools.partial(
    pl.kernel,
    out_type=jax.ShapeDtypeStruct((N_FLAT, EMBED_DIM), jnp.float32),
    mesh=plsc.VectorSubcoreMesh(core_axis_name="c", subcore_axis_name="s"),
    compiler_params=pltpu.CompilerParams(use_tc_tiling_on_sc=False),
    scratch_types=[
        pltpu.VMEM((IDX_ROWS, IDX_MINOR), jnp.int32),
        pltpu.VMEM((ROWS_PER_CHUNK, EMBED_DIM), jnp.float32),
        pltpu.VMEM((MAXLEN, EMBED_DIM), jnp.float32),
        pltpu.SemaphoreType.DMA,
    ],
)
def _embed_kernel(x_hbm, tok_hbm, pos_hbm, out_hbm, idx_v, rows_v, pos_v, sem):
    wid = lax.axis_index("s") * 2 + lax.axis_index("c")
    pltpu.sync_copy(pos_hbm, pos_v)

    def chunk_body(c, carry):
        chunk = wid * CHUNKS_PER_WORKER + c
        row0 = chunk * IDX_ROWS
        fr0 = chunk * ROWS_PER_CHUNK
        pltpu.sync_copy(x_hbm.at[pl.ds(row0, IDX_ROWS)], idx_v)
        copies = [
            pltpu.async_copy(
                tok_hbm.at[idx_v.at[j]],
                rows_v.at[pl.ds(j * IDX_MINOR, IDX_MINOR)],
                sem,
            )
            for j in range(IDX_ROWS)
        ]
        for cp in copies:
            cp.wait()

        def t_body(t, carry2):
            for p in range(EMBED_DIM // 16):
                pos_slice = pos_v[t, pl.ds(p * 16, 16)]
                for r in range(ROWS_PER_CHUNK // MAXLEN):
                    plsc.addupdate(
                        rows_v.at[r * MAXLEN + t, pl.ds(p * 16, 16)], pos_slice
                    )
            return carry2

        lax.fori_loop(0, MAXLEN, t_body, 0)
        pltpu.sync_copy(rows_v, out_hbm.at[pl.ds(fr0, ROWS_PER_CHUNK)])
        return carry

    lax.fori_loop(0, CHUNKS_PER_WORKER, chunk_body, 0)


def kernel(x, token_table, pos_table):
    x_flat = x.astype(jnp.int32).reshape(N_FLAT // IDX_MINOR, IDX_MINOR)
    out = _embed_kernel(x_flat, token_table, pos_table)
    return out.reshape(BATCH, MAXLEN, EMBED_DIM)
